# Initial kernel scaffold; baseline (speedup 1.0000x reference)
import jax
import jax.numpy as jnp
from jax.experimental import pallas as pl


def kernel(f0, f1, f2, f3, t0, t1, t2, t3):
    raise NotImplementedError("write your pallas kernel here")


# speedup vs baseline: 1.4378x; 1.4378x over previous
import jax
import jax.numpy as jnp
from jax import lax
from jax.experimental import pallas as pl
from jax.experimental.pallas import tpu as pltpu
from jax.experimental.pallas import tpu_sc as plsc

B = 4096
L = 20
DIM = 64
NF = 4
N = B * L
NC = 2
NS = 16
NW = NC * NS
ROWS_PER_W = N // NW
CHUNK = 128
NCHUNK = ROWS_PER_W // CHUNK
OUT_D = NF * DIM


def _body(i0, i1, i2, i3, oidx, t0, t1, t2, t3, out,
          idx_v, oidx_v, r0, r1, r2, r3, sem):
    c = lax.axis_index("c")
    s = lax.axis_index("s")
    wid = s * NC + c
    idxs = (i0, i1, i2, i3)
    tables = (t0, t1, t2, t3)
    rows = (r0, r1, r2, r3)
    for f in range(NF):
        pltpu.sync_copy(idxs[f].at[wid], idx_v.at[pl.ds(f * NCHUNK, NCHUNK)])
    pltpu.sync_copy(oidx.at[wid], oidx_v)

    def chunk_body(j, carry):
        cps = [
            pltpu.async_copy(
                tables[f].at[idx_v.at[f * NCHUNK + j]], rows[f], sem)
            for f in range(NF)
        ]
        for cp in cps:
            cp.wait()
        scs = [
            pltpu.async_copy(rows[f], out.at[oidx_v.at[f * NCHUNK + j]], sem)
            for f in range(NF)
        ]
        for cp in scs:
            cp.wait()
        return carry

    lax.fori_loop(0, NCHUNK, chunk_body, 0)


def kernel(f0, f1, f2, f3, t0, t1, t2, t3):
    mesh = plsc.VectorSubcoreMesh(core_axis_name="c", subcore_axis_name="s")
    run = pl.kernel(
        _body,
        mesh=mesh,
        compiler_params=pltpu.CompilerParams(use_tc_tiling_on_sc=False),
        out_type=jax.ShapeDtypeStruct((N * NF, DIM), jnp.float32),
        scratch_types=[
            pltpu.VMEM((NF * NCHUNK, CHUNK), jnp.int32),
            pltpu.VMEM((NF * NCHUNK, CHUNK), jnp.int32),
            pltpu.VMEM((CHUNK, DIM), jnp.float32),
            pltpu.VMEM((CHUNK, DIM), jnp.float32),
            pltpu.VMEM((CHUNK, DIM), jnp.float32),
            pltpu.VMEM((CHUNK, DIM), jnp.float32),
            pltpu.SemaphoreType.DMA,
        ],
    )
    idx = [x.reshape(NW, NCHUNK, CHUNK).astype(jnp.int32) for x in (f0, f1, f2, f3)]
    i_glob = jnp.arange(N, dtype=jnp.int32).reshape(NW, NCHUNK, CHUNK)
    oidx = jnp.concatenate(
        [NF * i_glob + f for f in range(NF)], axis=1)
    out = run(*idx, oidx, t0, t1, t2, t3)
    return out.reshape(B, L, OUT_D)


# speedup vs baseline: 1.4432x; 1.0038x over previous
import jax
import jax.numpy as jnp
from jax import lax
from jax.experimental import pallas as pl
from jax.experimental.pallas import tpu as pltpu
from jax.experimental.pallas import tpu_sc as plsc

B = 4096
L = 20
DIM = 64
NF = 4
N = B * L
NC = 2
NS = 16
NW = NC * NS
ROWS_PER_W = N // NW
CHUNK = 128
NCHUNK = ROWS_PER_W // CHUNK
OUT_D = NF * DIM


def _body(i0, i1, i2, i3, t0, t1, t2, t3, out, idx_v, comb_v, sem):
    c = lax.axis_index("c")
    s = lax.axis_index("s")
    wid = s * NC + c
    base = wid * ROWS_PER_W
    idxs = (i0, i1, i2, i3)
    tables = (t0, t1, t2, t3)
    for f in range(NF):
        pltpu.sync_copy(idxs[f].at[wid], idx_v.at[pl.ds(f * NCHUNK, NCHUNK)])

    def chunk_body(j, carry):
        cps = [
            pltpu.async_copy(
                tables[f].at[idx_v.at[f * NCHUNK + j]],
                comb_v.at[pl.ds(f * CHUNK, CHUNK)],
                sem,
            )
            for f in range(NF)
        ]
        for cp in cps:
            cp.wait()
        for f in range(NF):
            pltpu.sync_copy(
                comb_v.at[pl.ds(f * CHUNK, CHUNK)],
                out.at[pl.ds(base + j * CHUNK, CHUNK), pl.ds(f * DIM, DIM)],
            )
        return carry

    lax.fori_loop(0, NCHUNK, chunk_body, 0)


def kernel(f0, f1, f2, f3, t0, t1, t2, t3):
    mesh = plsc.VectorSubcoreMesh(core_axis_name="c", subcore_axis_name="s")
    run = pl.kernel(
        _body,
        mesh=mesh,
        compiler_params=pltpu.CompilerParams(use_tc_tiling_on_sc=False),
        out_type=jax.ShapeDtypeStruct((N, OUT_D), jnp.float32),
        scratch_types=[
            pltpu.VMEM((NF * NCHUNK, CHUNK), jnp.int32),
            pltpu.VMEM((NF * CHUNK, DIM), jnp.float32),
            pltpu.SemaphoreType.DMA,
        ],
    )
    idx = [x.reshape(NW, NCHUNK, CHUNK).astype(jnp.int32) for x in (f0, f1, f2, f3)]
    out = run(*idx, t0, t1, t2, t3)
    return out.reshape(B, L, OUT_D)
